# Initial kernel scaffold; baseline (speedup 1.0000x reference)
import jax
import jax.numpy as jnp
from jax.experimental import pallas as pl


def kernel(adj, features, W_base, W_mean, W_base2, W1_0, al1_0, ar1_0, b1_0, W1_1, al1_1, ar1_1, b1_1, W1_2, al1_2, ar1_2, b1_2, W2_0, al2_0, ar2_0, b2_0, W2_1, al2_1, ar2_1, b2_1, W2_2, al2_2, ar2_2, b2_2, W_score, b_score):
    raise NotImplementedError("write your pallas kernel here")


# speedup vs baseline: 3028.0592x; 3028.0592x over previous
import jax
import jax.numpy as jnp
from jax.experimental import pallas as pl

N = 1024
F32 = jnp.float32


def _dot(a, b):
    return jax.lax.dot_general(a, b, (((1,), (0,)), ((), ())),
                               preferred_element_type=F32)


def _fused_kernel(adj_ref, feat_ref, wb_ref, wm_ref, wb2_ref,
                  w1_ref, al1_ref, ar1_ref, b1_ref,
                  w2_ref, al2_ref, ar2_ref, b2_ref,
                  ws_ref, bs_ref, out_ref):
    adj = adj_ref[...]
    feats = feat_ref[...]

    t0 = _dot(feats, wb_ref[...])
    hidden = _dot(adj, t0)
    zp = _dot(adj, _dot(hidden, wm_ref[...]))
    z = jnp.where(zp > 0, zp, jnp.exp(zp) - 1.0)
    pred_x = _dot(adj, _dot(z, wb2_ref[...]))

    def attn_branch(x, al_row, ar_row, b_row):
        el = jnp.sum(x * al_row, axis=1, keepdims=True)
        er = jnp.sum(x * ar_row, axis=1, keepdims=True)
        s = el + er.T
        w = jnp.exp(jnp.where(s >= 0, s, 0.2 * s))
        num = _dot(w, x)
        den = jnp.maximum(jnp.sum(w, axis=1, keepdims=True), 1e-12)
        return num / den + b_row

    x1 = _dot(feats, w1_ref[...])
    h1 = attn_branch(x1, al1_ref[...], ar1_ref[...], b1_ref[...])
    x2 = _dot(pred_x, w2_ref[...])
    h2 = attn_branch(x2, al2_ref[...], ar2_ref[...], b2_ref[...])

    ws = ws_ref[...]
    logit = (jnp.sum(h1 * ws[:, :64], axis=1, keepdims=True)
             + jnp.sum(h2 * ws[:, 64:], axis=1, keepdims=True)
             + bs_ref[...])
    out_ref[...] = jax.nn.sigmoid(logit)


def kernel(adj, features, W_base, W_mean, W_base2,
           W1_0, al1_0, ar1_0, b1_0, W1_1, al1_1, ar1_1, b1_1,
           W1_2, al1_2, ar1_2, b1_2,
           W2_0, al2_0, ar2_0, b2_0, W2_1, al2_1, ar2_1, b2_1,
           W2_2, al2_2, ar2_2, b2_2, W_score, b_score):
    out = pl.pallas_call(
        _fused_kernel,
        out_shape=jax.ShapeDtypeStruct((N, 1), F32),
    )(adj, features, W_base, W_mean, W_base2,
      W1_2, al1_2.reshape(1, -1), ar1_2.reshape(1, -1), b1_2.reshape(1, -1),
      W2_2, al2_2.reshape(1, -1), ar2_2.reshape(1, -1), b2_2.reshape(1, -1),
      W_score.reshape(1, -1), b_score.reshape(1, 1))
    return out
